# Initial kernel scaffold; baseline (speedup 1.0000x reference)
import jax
import jax.numpy as jnp
from jax.experimental import pallas as pl


def kernel(inputs, target, negatives, emb_table, W1, b1, W2, b2):
    raise NotImplementedError("write your pallas kernel here")


# speedup vs baseline: 14.3310x; 14.3310x over previous
import functools

import jax
import jax.numpy as jnp
from jax import lax
from jax.experimental import pallas as pl
from jax.experimental.pallas import tpu as pltpu
from jax.experimental.pallas import tpu_sc as plsc

EMB = 32
WIN = 5
NEG = 20
B = 16384
ENT = 2 * WIN + 1 + NEG
N = ENT * B
NW = 32
GSZ = 128
NG = N // (NW * GSZ)

@functools.lru_cache(maxsize=1)
def _make_sc_gather():
    mesh = plsc.VectorSubcoreMesh(core_axis_name="c", subcore_axis_name="s")

    @functools.partial(
        pl.kernel,
        mesh=mesh,
        out_type=jax.ShapeDtypeStruct((NW * NG, GSZ, EMB), jnp.float32),
        scratch_types=[
            pltpu.VMEM((NG, GSZ), jnp.int32),
            pltpu.VMEM((GSZ, EMB), jnp.float32),
            pltpu.SemaphoreType.DMA,
        ],
        compiler_params=pltpu.CompilerParams(use_tc_tiling_on_sc=False),
    )
    def _sc_gather(table, idx, out, idx_v, rows_v, sem):
        wid = lax.axis_index("s") * 2 + lax.axis_index("c")
        pltpu.sync_copy(idx.at[wid], idx_v)

        def body(g, carry):
            pltpu.async_copy(table.at[idx_v.at[g]], rows_v, sem).wait()
            pltpu.sync_copy(rows_v, out.at[wid * NG + g])
            return carry

        lax.fori_loop(0, NG, body, 0)

    return _sc_gather


BLK = 512


def _tc_body(g_ref, w1_ref, b1_ref, w2_ref, b2_ref, o_ref):
    g = g_ref[...].reshape(ENT * BLK, EMB)
    h = jax.nn.sigmoid(
        jnp.dot(g, w1_ref[...], preferred_element_type=jnp.float32)
        + b1_ref[...]
    )
    h = h.reshape(ENT, BLK, EMB)
    means = h[0]
    for i in range(1, 2 * WIN):
        means = means + h[i]
    means = means * (1.0 / (2 * WIN))
    acc = jnp.dot(means * h[2 * WIN], w2_ref[0:EMB, :],
                  preferred_element_type=jnp.float32)
    for i in range(NEG):
        r = (i + 1) * EMB
        acc = acc + jnp.dot(means * h[2 * WIN + 1 + i], w2_ref[r:r + EMB, :],
                            preferred_element_type=jnp.float32)
    logits = acc + b2_ref[...]
    m = jnp.max(logits, axis=-1, keepdims=True)
    e = jnp.exp(logits - m)
    o_ref[...] = e / jnp.sum(e, axis=-1, keepdims=True)


def _tc_dense(g3, W1, b1, W2, b2):
    return pl.pallas_call(
        _tc_body,
        grid=(B // BLK,),
        in_specs=[
            pl.BlockSpec((ENT, BLK, EMB), lambda i: (0, i, 0)),
            pl.BlockSpec((EMB, EMB), lambda i: (0, 0)),
            pl.BlockSpec((1, EMB), lambda i: (0, 0)),
            pl.BlockSpec(((NEG + 1) * EMB, NEG + 1), lambda i: (0, 0)),
            pl.BlockSpec((1, NEG + 1), lambda i: (0, 0)),
        ],
        out_specs=pl.BlockSpec((BLK, NEG + 1), lambda i: (i, 0)),
        out_shape=jax.ShapeDtypeStruct((B, NEG + 1), jnp.float32),
    )(g3, W1, b1, W2, b2)


def kernel(inputs, target, negatives, emb_table, W1, b1, W2, b2):
    idx = jnp.concatenate([inputs, target, negatives], axis=1)
    idx = idx.astype(jnp.int32).T.reshape(NW, NG, GSZ)
    gathered = _make_sc_gather()(emb_table, idx)
    g3 = gathered.reshape(ENT, B, EMB)
    return _tc_dense(g3, W1, b1.reshape(1, EMB), W2, b2.reshape(1, NEG + 1))


# speedup vs baseline: 19.7212x; 1.3761x over previous
import functools

import jax
import jax.numpy as jnp
from jax import lax
from jax.experimental import pallas as pl
from jax.experimental.pallas import tpu as pltpu
from jax.experimental.pallas import tpu_sc as plsc

EMB = 32
WIN = 5
NEG = 20
B = 16384
ENT = 2 * WIN + 1 + NEG
NW = 32
GSZ = 128
NCH = ENT * (B // GSZ)
NG = NCH // NW
NB = 8
B4 = B // 4
NE = NEG + 1


def _tr_body(i_ref, o_ref):
    x = i_ref[...]
    o_ref[...] = x.T.reshape(ENT, 8, GSZ)


def _transpose_idx(idxcat):
    return pl.pallas_call(
        _tr_body,
        grid=(B // 1024,),
        in_specs=[pl.BlockSpec((1024, ENT), lambda j: (j, 0))],
        out_specs=pl.BlockSpec((ENT, 8, GSZ), lambda j: (0, j, 0)),
        out_shape=jax.ShapeDtypeStruct((ENT, B // GSZ, GSZ), jnp.int32),
    )(idxcat)


@functools.lru_cache(maxsize=1)
def _make_sc_gather():
    mesh = plsc.VectorSubcoreMesh(core_axis_name="c", subcore_axis_name="s")

    @functools.partial(
        pl.kernel,
        mesh=mesh,
        out_type=jax.ShapeDtypeStruct((ENT, B, EMB), jnp.float32),
        scratch_types=[
            pltpu.VMEM((NB, GSZ), jnp.int32),
            pltpu.VMEM((NB, GSZ, EMB), jnp.float32),
            pltpu.SemaphoreType.DMA((NB,)),
            pltpu.SemaphoreType.DMA((NB,)),
        ],
        compiler_params=pltpu.CompilerParams(use_tc_tiling_on_sc=False),
    )
    def _sc_gather(table, idx3, out, idx_v, rows_v, isem, gsem):
        wid = lax.axis_index("s") * 2 + lax.axis_index("c")
        c0 = wid * NG

        def idx_src(c):
            return idx3.at[c // GSZ, lax.rem(c, GSZ)]

        def fire_idx(c, b):
            pltpu.async_copy(idx_src(c), idx_v.at[b], isem.at[b])

        def wait_idx(c, b):
            pltpu.make_async_copy(idx_src(c), idx_v.at[b], isem.at[b]).wait()

        def fire_gather(b):
            pltpu.async_copy(table.at[idx_v.at[b]], rows_v.at[b], gsem.at[b])

        def wait_gather(b):
            pltpu.make_async_copy(
                table.at[idx_v.at[b]], rows_v.at[b], gsem.at[b]
            ).wait()

        def store(c, b):
            pltpu.sync_copy(
                rows_v.at[b],
                out.at[c // GSZ, pl.ds(lax.rem(c, GSZ) * GSZ, GSZ)],
            )

        for b in range(NB):
            fire_idx(c0 + b, b)
        for b in range(NB):
            wait_idx(c0 + b, b)
            fire_gather(b)

        def outer(o, carry):
            for b in range(NB):
                r = o * NB + b
                c = c0 + r
                wait_gather(b)
                store(c, b)

                @pl.when(r + NB < NG)
                def _refill():
                    fire_idx(c + NB, b)
                    wait_idx(c + NB, b)
                    fire_gather(b)

            return carry

        lax.fori_loop(0, NG // NB, outer, 0)

        rem = NG - (NG // NB) * NB
        for b in range(rem):
            c = c0 + (NG // NB) * NB + b
            wait_gather(b)
            store(c, b)

    return _sc_gather


BQ = 128


def _tc_body(g_ref, w1_ref, b1_ref, w2_ref, b2_ref, o_ref):
    g = g_ref[...].reshape(ENT * BQ, 128)
    h = jax.nn.sigmoid(
        jnp.dot(g, w1_ref[...], preferred_element_type=jnp.float32)
        + b1_ref[...]
    )
    h = h.reshape(ENT, BQ, 128)
    means = h[0]
    for i in range(1, 2 * WIN):
        means = means + h[i]
    means = means * (1.0 / (2 * WIN))
    acc = jnp.dot(means * h[2 * WIN], w2_ref[0],
                  preferred_element_type=jnp.float32)
    for i in range(1, NE):
        acc = acc + jnp.dot(means * h[2 * WIN + i], w2_ref[i],
                            preferred_element_type=jnp.float32)
    logits = acc + b2_ref[...]
    for k in range(4):
        seg = logits[:, NE * k:NE * (k + 1)]
        m = jnp.max(seg, axis=-1, keepdims=True)
        ex = jnp.exp(seg - m)
        o_ref[:, NE * k:NE * (k + 1)] = ex / jnp.sum(ex, axis=-1,
                                                     keepdims=True)


def _tc_dense(g4, W1bd, b1t, W2p, b2t):
    return pl.pallas_call(
        _tc_body,
        grid=(B4 // BQ,),
        in_specs=[
            pl.BlockSpec((ENT, BQ, 128), lambda i: (0, i, 0)),
            pl.BlockSpec((128, 128), lambda i: (0, 0)),
            pl.BlockSpec((1, 128), lambda i: (0, 0)),
            pl.BlockSpec((NE, 128, 4 * NE), lambda i: (0, 0, 0)),
            pl.BlockSpec((1, 4 * NE), lambda i: (0, 0)),
        ],
        out_specs=pl.BlockSpec((BQ, 4 * NE), lambda i: (i, 0)),
        out_shape=jax.ShapeDtypeStruct((B4, 4 * NE), jnp.float32),
    )(g4, W1bd, b1t, W2p, b2t)


def kernel(inputs, target, negatives, emb_table, W1, b1, W2, b2):
    idxcat = jnp.concatenate([inputs, target, negatives], axis=1)
    idxcat = idxcat.astype(jnp.int32)
    idx3 = _transpose_idx(idxcat)
    gathered = _make_sc_gather()(emb_table, idx3)
    g4 = gathered.reshape(ENT, B4, GSZ)

    eye4 = jnp.eye(4, dtype=jnp.float32)
    W1bd = jnp.kron(eye4, W1)
    b1t = jnp.tile(b1, 4).reshape(1, 128)
    W2p = jnp.stack(
        [jnp.kron(eye4, W2[i * EMB:(i + 1) * EMB, :]) for i in range(NE)]
    )
    b2t = jnp.tile(b2, 4).reshape(1, 4 * NE)

    out84 = _tc_dense(g4, W1bd, b1t, W2p, b2t)
    return out84.reshape(B, NE)


# speedup vs baseline: 29.8996x; 1.5161x over previous
import functools

import jax
import jax.numpy as jnp
from jax import lax
from jax.experimental import pallas as pl
from jax.experimental.pallas import tpu as pltpu
from jax.experimental.pallas import tpu_sc as plsc

EMB = 32
WIN = 5
NEG = 20
B = 16384
ENT = 2 * WIN + 1 + NEG
NW = 32
GSZ = 128
NCH = ENT * (B // GSZ)
NG = NCH // NW
NB = 8
B4 = B // 4
NE = NEG + 1


def _tr_body(i_ref, o_ref):
    v = i_ref[...]
    g = (v & -8192) + ((v & 2047) << 2) + ((v & 8191) >> 11)
    o_ref[...] = g.T.reshape(ENT, 8, GSZ)


def _transpose_idx(idxcat):
    return pl.pallas_call(
        _tr_body,
        grid=(B // 1024,),
        in_specs=[pl.BlockSpec((1024, ENT), lambda j: (j, 0))],
        out_specs=pl.BlockSpec((ENT, 8, GSZ), lambda j: (0, j, 0)),
        out_shape=jax.ShapeDtypeStruct((ENT, B // GSZ, GSZ), jnp.int32),
    )(idxcat)


VOCAB = 1000000
RB = 8192
NRB = (VOCAB + RB - 1) // RB
TROWS = NRB * RB


def _rp_body(t_ref, o_ref):
    x = t_ref[...]
    q = RB // 4
    parts = [x[:, a * q:(a + 1) * q].T for a in range(4)]
    o_ref[...] = jnp.concatenate(parts, axis=1)


def _repack_table(tT):
    return pl.pallas_call(
        _rp_body,
        grid=(NRB,),
        in_specs=[pl.BlockSpec((EMB, RB), lambda j: (0, j))],
        out_specs=pl.BlockSpec((RB // 4, 128), lambda j: (j, 0)),
        out_shape=jax.ShapeDtypeStruct((TROWS // 4, 128), jnp.float32),
    )(tT)


@functools.lru_cache(maxsize=1)
def _make_sc_gather():
    mesh = plsc.VectorSubcoreMesh(core_axis_name="c", subcore_axis_name="s")

    @functools.partial(
        pl.kernel,
        mesh=mesh,
        out_type=jax.ShapeDtypeStruct((ENT, B, EMB), jnp.float32),
        scratch_types=[
            pltpu.VMEM((NB, GSZ), jnp.int32),
            pltpu.VMEM((NB, GSZ, EMB), jnp.float32),
            pltpu.SemaphoreType.DMA((NB,)),
            pltpu.SemaphoreType.DMA((NB,)),
        ],
        compiler_params=pltpu.CompilerParams(use_tc_tiling_on_sc=False),
    )
    def _sc_gather(table, idx3, out, idx_v, rows_v, isem, gsem):
        wid = lax.axis_index("s") * 2 + lax.axis_index("c")
        c0 = wid * NG

        def idx_src(c):
            return idx3.at[c // GSZ, lax.rem(c, GSZ)]

        def fire_idx(c, b):
            pltpu.async_copy(idx_src(c), idx_v.at[b], isem.at[b])

        def wait_idx(c, b):
            pltpu.make_async_copy(idx_src(c), idx_v.at[b], isem.at[b]).wait()

        def fire_gather(b):
            pltpu.async_copy(table.at[idx_v.at[b]], rows_v.at[b], gsem.at[b])

        def wait_gather(b):
            pltpu.make_async_copy(
                table.at[idx_v.at[b]], rows_v.at[b], gsem.at[b]
            ).wait()

        def store(c, b):
            pltpu.sync_copy(
                rows_v.at[b],
                out.at[c // GSZ, pl.ds(lax.rem(c, GSZ) * GSZ, GSZ)],
            )

        for b in range(NB):
            fire_idx(c0 + b, b)
        for b in range(NB):
            wait_idx(c0 + b, b)
            fire_gather(b)

        def outer(o, carry):
            for b in range(NB):
                r = o * NB + b
                c = c0 + r
                wait_gather(b)
                store(c, b)

                @pl.when(r + NB < NG)
                def _refill():
                    fire_idx(c + NB, b)
                    wait_idx(c + NB, b)
                    fire_gather(b)

            return carry

        lax.fori_loop(0, NG // NB, outer, 0)

        rem = NG - (NG // NB) * NB
        for b in range(rem):
            c = c0 + (NG // NB) * NB + b
            wait_gather(b)
            store(c, b)

    return _sc_gather


BQ = 128


def _tc_body(g_ref, w1_ref, b1_ref, w2_ref, b2_ref, o_ref):
    g = g_ref[...].reshape(ENT * BQ, 128)
    h = jax.nn.sigmoid(
        jnp.dot(g, w1_ref[...], preferred_element_type=jnp.float32)
        + b1_ref[...]
    )
    h = h.reshape(ENT, BQ, 128)
    means = h[0]
    for i in range(1, 2 * WIN):
        means = means + h[i]
    means = means * (1.0 / (2 * WIN))
    acc = jnp.dot(means * h[2 * WIN], w2_ref[0],
                  preferred_element_type=jnp.float32)
    for i in range(1, NE):
        acc = acc + jnp.dot(means * h[2 * WIN + i], w2_ref[i],
                            preferred_element_type=jnp.float32)
    logits = acc + b2_ref[...]
    for k in range(4):
        seg = logits[:, NE * k:NE * (k + 1)]
        m = jnp.max(seg, axis=-1, keepdims=True)
        ex = jnp.exp(seg - m)
        o_ref[:, NE * k:NE * (k + 1)] = ex / jnp.sum(ex, axis=-1,
                                                     keepdims=True)


def _tc_dense(g4, W1bd, b1t, W2p, b2t):
    return pl.pallas_call(
        _tc_body,
        grid=(B4 // BQ,),
        in_specs=[
            pl.BlockSpec((ENT, BQ, 128), lambda i: (0, i, 0)),
            pl.BlockSpec((128, 128), lambda i: (0, 0)),
            pl.BlockSpec((1, 128), lambda i: (0, 0)),
            pl.BlockSpec((NE, 128, 4 * NE), lambda i: (0, 0, 0)),
            pl.BlockSpec((1, 4 * NE), lambda i: (0, 0)),
        ],
        out_specs=pl.BlockSpec((BQ, 4 * NE), lambda i: (i, 0)),
        out_shape=jax.ShapeDtypeStruct((B4, 4 * NE), jnp.float32),
    )(g4, W1bd, b1t, W2p, b2t)


def kernel(inputs, target, negatives, emb_table, W1, b1, W2, b2):
    idxcat = jnp.concatenate([inputs, target, negatives], axis=1)
    idxcat = idxcat.astype(jnp.int32)
    idx3 = _transpose_idx(idxcat)
    table_flat = _repack_table(emb_table.T).reshape(TROWS, EMB)
    gathered = _make_sc_gather()(table_flat, idx3)
    g4 = gathered.reshape(ENT, B4, GSZ)

    eye4 = jnp.eye(4, dtype=jnp.float32)
    W1bd = jnp.kron(eye4, W1)
    b1t = jnp.tile(b1, 4).reshape(1, 128)
    W2p = jnp.stack(
        [jnp.kron(eye4, W2[i * EMB:(i + 1) * EMB, :]) for i in range(NE)]
    )
    b2t = jnp.tile(b2, 4).reshape(1, 4 * NE)

    out84 = _tc_dense(g4, W1bd, b1t, W2p, b2t)
    return out84.reshape(B, NE)
